# baseline (speedup 1.0000x reference)
import functools

import jax
import jax.numpy as jnp
from jax import lax
from jax.experimental import pallas as pl
from jax.experimental.pallas import tpu as pltpu
from jax.experimental.pallas import tpu_sc as plsc

D = 64
B = 16384
NMAT = 9
NIDX = NMAT * B
NW = 32
CHUNK = 128
CPW = NIDX // (NW * CHUNK)
MARGIN = 0.1
EPS = 1e-5


PBC = B // 2


_SRC_OF = {0: (2, 0), 1: (2, 1), 2: (3, 0), 3: (3, 1), 4: (3, 2),
           5: (4, 1), 6: (4, 2), 7: (5, 0), 8: (5, 2)}


def _sc_rad_gather(go_rad_flat, nf1t, nf2t, nf3t, nf4t):
    @functools.partial(
        pl.kernel,
        out_type=jax.ShapeDtypeStruct((NMAT * B,), jnp.float32),
        mesh=plsc.VectorSubcoreMesh(core_axis_name="c", subcore_axis_name="s"),
        compiler_params=pltpu.CompilerParams(use_tc_tiling_on_sc=False),
        scratch_types=[
            pltpu.VMEM((CHUNK,), jnp.int32),
            pltpu.VMEM((CHUNK,), jnp.int32),
            pltpu.VMEM((CHUNK,), jnp.float32),
            pltpu.VMEM((CHUNK,), jnp.float32),
            pltpu.SemaphoreType.DMA,
            pltpu.SemaphoreType.DMA,
        ],
    )
    def rad_kernel(rad_hbm, n1_hbm, n2_hbm, n3_hbm, n4_hbm, rad_out,
                   idx0, idx1, radv0, radv1, sr0, sr1):
        wid = lax.axis_index("s") * 2 + lax.axis_index("c")
        refs = (None, rad_hbm, n1_hbm, n2_hbm, n3_hbm, n4_hbm)
        slots = ((idx0, radv0, sr0), (idx1, radv1, sr1))
        njobs = 4 * NMAT

        def job(j):
            m, t = j // 4, j % 4
            h, tt = t // 2, t % 2
            return m, h, wid * 2 + tt

        def issue(j, slot):
            idxb, radb, sr = slot
            m, h, c = job(j)
            pos, col = _SRC_OF[m]
            pltpu.sync_copy(
                refs[pos].at[col, pl.ds(h * PBC + c * CHUNK, CHUNK)], idxb)
            return pltpu.async_copy(rad_hbm.at[idxb], radb, sr)

        def drain(j, slot, cp):
            idxb, radb, sr = slot
            cp.wait()
            m, h, c = job(j)
            pltpu.sync_copy(
                radb,
                rad_out.at[pl.ds(m * B + h * PBC + c * CHUNK, CHUNK)])

        cp = issue(0, slots[0])
        for j in range(njobs):
            nxt = issue(j + 1, slots[(j + 1) % 2]) if j + 1 < njobs else None
            drain(j, slots[j % 2], cp)
            cp = nxt

    return rad_kernel(go_rad_flat, nf1t, nf2t, nf3t, nf4t)


def _sc_gather(emb128, nf1t, nf2t, nf3t, nf4t, kset):
    nm = len(kset)

    @functools.partial(
        pl.kernel,
        out_type=jax.ShapeDtypeStruct((nm * PBC, 2 * D), jnp.float32),
        mesh=plsc.VectorSubcoreMesh(core_axis_name="c", subcore_axis_name="s"),
        compiler_params=pltpu.CompilerParams(use_tc_tiling_on_sc=False),
        scratch_types=[
            pltpu.VMEM((CHUNK,), jnp.int32),
            pltpu.VMEM((CHUNK,), jnp.int32),
            pltpu.VMEM((CHUNK, 2 * D), jnp.float32),
            pltpu.VMEM((CHUNK, 2 * D), jnp.float32),
            pltpu.SemaphoreType.DMA,
            pltpu.SemaphoreType.DMA,
        ],
    )
    def gather_kernel(emb_hbm, n1_hbm, n2_hbm, n3_hbm, n4_hbm, rows_out,
                      idx0, idx1, rows0, rows1, sg0, sg1):
        wid = lax.axis_index("s") * 2 + lax.axis_index("c")
        refs = (None, None, n1_hbm, n2_hbm, n3_hbm, n4_hbm)
        slots = ((idx0, rows0, sg0), (idx1, rows1, sg1))
        njobs = 4 * nm

        def job(j):
            m, t = j // 4, j % 4
            h, tt = t // 2, t % 2
            return m, h, wid * 2 + tt

        def issue(j, slot):
            idxb, rowsb, sg = slot
            m, h, c = job(j)
            pos, col = _SRC_OF[kset[m]]
            pltpu.sync_copy(
                refs[pos].at[col, pl.ds(h * PBC + c * CHUNK, CHUNK)], idxb)
            return pltpu.async_copy(emb_hbm.at[idxb], rowsb, sg)

        def drain(j, slot, cp):
            idxb, rowsb, sg = slot
            cp.wait()
            m, h, c = job(j)
            pltpu.sync_copy(
                rowsb.at[pl.ds(0, CHUNK), pl.ds(0, D)],
                rows_out.at[pl.ds(m * PBC + c * CHUNK, CHUNK),
                            pl.ds(h * D, D)])

        cp = issue(0, slots[0])
        for j in range(njobs):
            nxt = issue(j + 1, slots[(j + 1) % 2]) if j + 1 < njobs else None
            drain(j, slots[j % 2], cp)
            cp = nxt

    return gather_kernel(emb128, nf1t, nf2t, nf3t, nf4t)


PB = B // 2
NBLK = 8
RP = PB // NBLK


def _stats_kernel(p_ref, g_ref, b_ref, out_ref):
    x = p_ref[:, :]
    s = jnp.sum(x, axis=0, keepdims=True)
    q = jnp.sum(x * x, axis=0, keepdims=True)
    sd = s[:, 0:D] + s[:, D:2 * D]
    qd = q[:, 0:D] + q[:, D:2 * D]
    m = sd * (1.0 / B)
    v = qd * (1.0 / B) - m * m
    sc = lax.rsqrt(v + EPS) * g_ref[:, :]
    sh = b_ref[:, :] - m * sc
    sc128 = jnp.concatenate([sc, sc], axis=1)
    sh128 = jnp.concatenate([sh, sh], axis=1)
    out_ref[0, :, :] = jnp.concatenate([sc128, sh128], axis=1)


def _relu(x):
    return jnp.maximum(x, 0.0)


NA = 5
NB_ = NMAT - NA


def _half_w():
    ii = lax.broadcasted_iota(jnp.int32, (128, 2), 0)
    jj = lax.broadcasted_iota(jnp.int32, (128, 2), 1)
    return jnp.where((ii < D) == (jj == 0), 1.0, 0.0).astype(jnp.float32)


def _bnx(mat, ss_ref, kk):
    scsh = ss_ref[kk:kk + 1, :]
    return mat[:, :] * scsh[:, 0:128] + scsh[:, 128:256]


def _accum(out_ref, i, part):
    @pl.when(i == 0)
    def _():
        out_ref[:, :] = jnp.zeros((1, 1), jnp.float32)

    out_ref[:, :] += jnp.reshape(part * (1.0 / B), (1, 1))


def _loss_a_kernel(m0, m1, m2, m3, m4, ss_ref, rads_ref, out_ref):
    i = pl.program_id(0)
    half_w = _half_w()

    def rnorm(dd):
        return jnp.sqrt(jnp.dot(dd * dd, half_w,
                                preferred_element_type=jnp.float32))

    def radp(k):
        return jnp.abs(rads_ref[k, :, :])

    part = jnp.sum(_relu(rnorm(_bnx(m0, ss_ref, 0) - _bnx(m1, ss_ref, 1))
                         + radp(0) - radp(1) - MARGIN))
    c = _bnx(m2, ss_ref, 2)
    d = _bnx(m3, ss_ref, 3)
    e = _bnx(m4, ss_ref, 4)
    rc = radp(2)
    rd = radp(3)
    part += jnp.sum(_relu(rnorm(c - d) - (rc + rd) - MARGIN)
                    + _relu(rnorm(e - c) - rc - MARGIN)
                    + _relu(rnorm(e - d) - rd - MARGIN))
    _accum(out_ref, i, part)


def _loss_b_kernel(m5, m6, m7, m8, ss_ref, rads_ref, ri3_ref, ri4_ref,
                   rel_ref, rad16_ref, out_ref):
    i = pl.program_id(0)
    half_w = _half_w()

    def rnorm(dd):
        return jnp.sqrt(jnp.dot(dd * dd, half_w,
                                preferred_element_type=jnp.float32))

    def radp(k):
        return jnp.abs(rads_ref[k, :, :])

    i16 = lax.broadcasted_iota(jnp.int32, (RP, 16), 1)
    f32 = jnp.float32
    oh3 = jnp.concatenate([(i16 == ri3_ref[:, 0:1]).astype(f32),
                           (i16 == ri3_ref[:, 1:2]).astype(f32)], axis=1)
    oh4 = jnp.concatenate([(i16 == ri4_ref[:, 0:1]).astype(f32),
                           (i16 == ri4_ref[:, 1:2]).astype(f32)], axis=1)
    z = jnp.zeros((16, D), f32)
    relblk = jnp.concatenate(
        [jnp.concatenate([rel_ref[:, :], z], axis=1),
         jnp.concatenate([z, rel_ref[:, :]], axis=1)], axis=0)
    rE3 = jnp.dot(oh3, relblk, preferred_element_type=f32)
    rE4 = jnp.dot(oh4, relblk, preferred_element_type=f32)
    a16 = jnp.abs(rad16_ref[:, :])
    z16 = jnp.zeros((16, 1), f32)
    w4 = jnp.concatenate([jnp.concatenate([a16, z16], axis=0),
                          jnp.concatenate([z16, a16], axis=0)], axis=1)
    rc4 = jnp.dot(oh4, w4, preferred_element_type=f32)

    part = jnp.sum(_relu(rnorm(_bnx(m5, ss_ref, 0) + rE3
                               - _bnx(m6, ss_ref, 1))
                         + radp(0) - radp(1) - MARGIN))
    part += jnp.sum(_relu(rnorm(_bnx(m7, ss_ref, 2)
                                - (_bnx(m8, ss_ref, 3) + rE4))
                          - (rc4 + radp(3)) - MARGIN))
    _accum(out_ref, i, part)


def _stats(packed, nm, gamma, beta):
    out = pl.pallas_call(
        _stats_kernel,
        grid=(nm,),
        out_shape=jax.ShapeDtypeStruct((nm, 1, 4 * D), jnp.float32),
        in_specs=[pl.BlockSpec((PB, 2 * D), lambda k: (k, 0)),
                  pl.BlockSpec((1, D), lambda k: (0, 0)),
                  pl.BlockSpec((1, D), lambda k: (0, 0))],
        out_specs=pl.BlockSpec((1, 1, 4 * D), lambda k: (k, 0, 0)),
    )(packed, gamma, beta)
    return out.reshape(nm, 4 * D)


def _mat_spec(kk):
    return pl.BlockSpec((RP, 2 * D), lambda i, kk=kk: (kk * NBLK + i, 0))


def _full_spec(shape):
    nd = len(shape)
    return pl.BlockSpec(shape, lambda i: (0,) * nd)


def _loss_a(packed_a, ssa, rads_a):
    return pl.pallas_call(
        _loss_a_kernel,
        grid=(NBLK,),
        out_shape=jax.ShapeDtypeStruct((1, 1), jnp.float32),
        in_specs=[_mat_spec(k) for k in range(NA)] + [
            _full_spec(ssa.shape),
            pl.BlockSpec((NA, RP, 2), lambda i: (0, i, 0)),
        ],
        out_specs=_full_spec((1, 1)),
    )(packed_a, packed_a, packed_a, packed_a, packed_a, ssa, rads_a)


def _loss_b(packed_b, ssb, rads_b, ri3p, ri4p, rel_embed, rad16):
    return pl.pallas_call(
        _loss_b_kernel,
        grid=(NBLK,),
        out_shape=jax.ShapeDtypeStruct((1, 1), jnp.float32),
        in_specs=[_mat_spec(k) for k in range(NB_)] + [
            _full_spec(ssb.shape),
            pl.BlockSpec((NB_, RP, 2), lambda i: (0, i, 0)),
            pl.BlockSpec((RP, 2), lambda i: (i, 0)),
            pl.BlockSpec((RP, 2), lambda i: (i, 0)),
            _full_spec(rel_embed.shape),
            _full_spec(rad16.shape),
        ],
        out_specs=_full_spec((1, 1)),
    )(packed_b, packed_b, packed_b, packed_b, ssb, rads_b,
      ri3p, ri4p, rel_embed, rad16)


def _tc_loss(packed_a, packed_b, rads_a, rads_b, ri3p, ri4p,
             rel_embed, rad16, gamma, beta):
    ssa = _stats(packed_a, NA, gamma, beta)
    la = _loss_a(packed_a, ssa, rads_a)
    ssb = _stats(packed_b, NB_, gamma, beta)
    lb = _loss_b(packed_b, ssb, rads_b, ri3p, ri4p, rel_embed, rad16)
    return la + lb


def kernel(nf1, nf2, nf3, nf4, go_embed, go_rad, rel_embed, bn_gamma, bn_beta):
    nf1 = nf1.astype(jnp.int32)
    nf2 = nf2.astype(jnp.int32)
    nf3 = nf3.astype(jnp.int32)
    nf4 = nf4.astype(jnp.int32)
    nf1t, nf2t, nf3t, nf4t = nf1.T, nf2.T, nf3.T, nf4.T
    rad_g = _sc_rad_gather(go_rad.reshape(-1), nf1t, nf2t, nf3t, nf4t)
    emb128 = jnp.pad(go_embed, ((0, 0), (0, D)))
    packed_a = _sc_gather(emb128, nf1t, nf2t, nf3t, nf4t,
                          kset=(0, 1, 2, 3, 4))
    packed_b = _sc_gather(emb128, nf1t, nf2t, nf3t, nf4t,
                          kset=(5, 6, 7, 8))
    rads_a = rad_g[:NA * B].reshape(NA, 2, PB).transpose(0, 2, 1)
    rads_b = rad_g[NA * B:].reshape(NB_, 2, PB).transpose(0, 2, 1)
    loss = _tc_loss(packed_a, packed_b, rads_a, rads_b,
                    nf3t[0].reshape(2, PB).T, nf4t[1].reshape(2, PB).T,
                    rel_embed, go_rad[:16],
                    bn_gamma.reshape(1, D), bn_beta.reshape(1, D))
    return loss[0, 0]


# speedup vs baseline: 1.0763x; 1.0763x over previous
import functools

import jax
import jax.numpy as jnp
from jax import lax
from jax.experimental import pallas as pl
from jax.experimental.pallas import tpu as pltpu
from jax.experimental.pallas import tpu_sc as plsc

D = 64
B = 16384
NMAT = 9
NIDX = NMAT * B
NW = 32
CHUNK = 128
CPW = NIDX // (NW * CHUNK)
MARGIN = 0.1
EPS = 1e-5


PBC = B // 2


def _sc_gather(go_embed, go_rad_flat, nf1t, nf2t, nf3t, nf4t, kset):
    nm = len(kset)
    src_of = {0: (2, 0), 1: (2, 1), 2: (3, 0), 3: (3, 1), 4: (3, 2),
              5: (4, 1), 6: (4, 2), 7: (5, 0), 8: (5, 2)}

    @functools.partial(
        pl.kernel,
        out_type=[
            jax.ShapeDtypeStruct((nm * PBC, 2 * D), jnp.float32),
            jax.ShapeDtypeStruct((nm * B,), jnp.float32),
        ],
        mesh=plsc.VectorSubcoreMesh(core_axis_name="c", subcore_axis_name="s"),
        compiler_params=pltpu.CompilerParams(use_tc_tiling_on_sc=False),
        scratch_types=[
            pltpu.VMEM((CHUNK,), jnp.int32),
            pltpu.VMEM((CHUNK,), jnp.int32),
            pltpu.VMEM((CHUNK, D), jnp.float32),
            pltpu.VMEM((CHUNK, D), jnp.float32),
            pltpu.VMEM((CHUNK,), jnp.float32),
            pltpu.VMEM((CHUNK,), jnp.float32),
            pltpu.SemaphoreType.DMA,
            pltpu.SemaphoreType.DMA,
            pltpu.SemaphoreType.DMA,
            pltpu.SemaphoreType.DMA,
        ],
    )
    def gather_kernel(emb_hbm, rad_hbm, n1_hbm, n2_hbm, n3_hbm, n4_hbm,
                      rows_out, rad_out,
                      idx0, idx1, rows0, rows1, radv0, radv1,
                      sg0, sg1, sr0, sr1):
        wid = lax.axis_index("s") * 2 + lax.axis_index("c")
        refs = (emb_hbm, rad_hbm, n1_hbm, n2_hbm, n3_hbm, n4_hbm)
        slots = ((idx0, rows0, radv0, sg0, sr0),
                 (idx1, rows1, radv1, sg1, sr1))
        njobs = 4 * nm

        def job(j):
            m, t = j // 4, j % 4
            h, tt = t // 2, t % 2
            return m, h, wid * 2 + tt

        def issue(j, slot):
            idxb, rowsb, radb, sg, sr = slot
            m, h, c = job(j)
            pos, col = src_of[kset[m]]
            pltpu.sync_copy(
                refs[pos].at[col, pl.ds(h * PBC + c * CHUNK, CHUNK)], idxb)
            cp_r = pltpu.async_copy(emb_hbm.at[idxb], rowsb, sg)
            cp_d = pltpu.async_copy(rad_hbm.at[idxb], radb, sr)
            return cp_r, cp_d

        def drain(j, slot, cps):
            idxb, rowsb, radb, sg, sr = slot
            cps[0].wait()
            cps[1].wait()
            m, h, c = job(j)
            pltpu.sync_copy(
                rowsb,
                rows_out.at[pl.ds(m * PBC + c * CHUNK, CHUNK),
                            pl.ds(h * D, D)])
            pltpu.sync_copy(
                radb,
                rad_out.at[pl.ds(m * B + h * PBC + c * CHUNK, CHUNK)])

        cps = issue(0, slots[0])
        for j in range(njobs):
            nxt = issue(j + 1, slots[(j + 1) % 2]) if j + 1 < njobs else None
            drain(j, slots[j % 2], cps)
            cps = nxt

    return gather_kernel(go_embed, go_rad_flat, nf1t, nf2t, nf3t, nf4t)


def _radt_kernel(in_ref, out_ref):
    out_ref[0, :, :] = jnp.transpose(in_ref[0, :, :], (1, 0))


def _rad_transpose(rad_flat, nm):
    return pl.pallas_call(
        _radt_kernel,
        grid=(nm,),
        out_shape=jax.ShapeDtypeStruct((nm, PBC, 2), jnp.float32),
        in_specs=[pl.BlockSpec((1, 2, PBC), lambda k: (k, 0, 0))],
        out_specs=pl.BlockSpec((1, PBC, 2), lambda k: (k, 0, 0)),
    )(rad_flat.reshape(nm, 2, PBC))


PB = B // 2
NBLK = 8
RP = PB // NBLK


def _stats_kernel(p_ref, g_ref, b_ref, out_ref):
    x = p_ref[:, :]
    s = jnp.sum(x, axis=0, keepdims=True)
    q = jnp.sum(x * x, axis=0, keepdims=True)
    sd = s[:, 0:D] + s[:, D:2 * D]
    qd = q[:, 0:D] + q[:, D:2 * D]
    m = sd * (1.0 / B)
    v = qd * (1.0 / B) - m * m
    sc = lax.rsqrt(v + EPS) * g_ref[:, :]
    sh = b_ref[:, :] - m * sc
    sc128 = jnp.concatenate([sc, sc], axis=1)
    sh128 = jnp.concatenate([sh, sh], axis=1)
    out_ref[0, :, :] = jnp.concatenate([sc128, sh128], axis=1)


def _relu(x):
    return jnp.maximum(x, 0.0)


NA = 5
NB_ = NMAT - NA


def _half_w():
    ii = lax.broadcasted_iota(jnp.int32, (128, 2), 0)
    jj = lax.broadcasted_iota(jnp.int32, (128, 2), 1)
    return jnp.where((ii < D) == (jj == 0), 1.0, 0.0).astype(jnp.float32)


def _bnx(mat, ss_ref, kk):
    scsh = ss_ref[kk:kk + 1, :]
    return mat[:, :] * scsh[:, 0:128] + scsh[:, 128:256]


def _accum(out_ref, i, part):
    @pl.when(i == 0)
    def _():
        out_ref[:, :] = jnp.zeros((1, 1), jnp.float32)

    out_ref[:, :] += jnp.reshape(part * (1.0 / B), (1, 1))


def _loss_a_kernel(m0, m1, m2, m3, m4, ss_ref, rads_ref, out_ref):
    i = pl.program_id(0)
    half_w = _half_w()

    def rnorm(dd):
        return jnp.sqrt(jnp.dot(dd * dd, half_w,
                                preferred_element_type=jnp.float32))

    def radp(k):
        return jnp.abs(rads_ref[k, :, :])

    part = jnp.sum(_relu(rnorm(_bnx(m0, ss_ref, 0) - _bnx(m1, ss_ref, 1))
                         + radp(0) - radp(1) - MARGIN))
    c = _bnx(m2, ss_ref, 2)
    d = _bnx(m3, ss_ref, 3)
    e = _bnx(m4, ss_ref, 4)
    rc = radp(2)
    rd = radp(3)
    part += jnp.sum(_relu(rnorm(c - d) - (rc + rd) - MARGIN)
                    + _relu(rnorm(e - c) - rc - MARGIN)
                    + _relu(rnorm(e - d) - rd - MARGIN))
    _accum(out_ref, i, part)


def _loss_b_kernel(m5, m6, m7, m8, ss_ref, rads_ref, ri3_ref, ri4_ref,
                   rel_ref, rad16_ref, out_ref):
    i = pl.program_id(0)
    half_w = _half_w()

    def rnorm(dd):
        return jnp.sqrt(jnp.dot(dd * dd, half_w,
                                preferred_element_type=jnp.float32))

    def radp(k):
        return jnp.abs(rads_ref[k, :, :])

    i16 = lax.broadcasted_iota(jnp.int32, (RP, 16), 1)
    f32 = jnp.float32
    oh3 = jnp.concatenate([(i16 == ri3_ref[:, 0:1]).astype(f32),
                           (i16 == ri3_ref[:, 1:2]).astype(f32)], axis=1)
    oh4 = jnp.concatenate([(i16 == ri4_ref[:, 0:1]).astype(f32),
                           (i16 == ri4_ref[:, 1:2]).astype(f32)], axis=1)
    z = jnp.zeros((16, D), f32)
    relblk = jnp.concatenate(
        [jnp.concatenate([rel_ref[:, :], z], axis=1),
         jnp.concatenate([z, rel_ref[:, :]], axis=1)], axis=0)
    rE3 = jnp.dot(oh3, relblk, preferred_element_type=f32)
    rE4 = jnp.dot(oh4, relblk, preferred_element_type=f32)
    a16 = jnp.abs(rad16_ref[:, :])
    z16 = jnp.zeros((16, 1), f32)
    w4 = jnp.concatenate([jnp.concatenate([a16, z16], axis=0),
                          jnp.concatenate([z16, a16], axis=0)], axis=1)
    rc4 = jnp.dot(oh4, w4, preferred_element_type=f32)

    part = jnp.sum(_relu(rnorm(_bnx(m5, ss_ref, 0) + rE3
                               - _bnx(m6, ss_ref, 1))
                         + radp(0) - radp(1) - MARGIN))
    part += jnp.sum(_relu(rnorm(_bnx(m7, ss_ref, 2)
                                - (_bnx(m8, ss_ref, 3) + rE4))
                          - (rc4 + radp(3)) - MARGIN))
    _accum(out_ref, i, part)


def _stats(packed, nm, gamma, beta):
    out = pl.pallas_call(
        _stats_kernel,
        grid=(nm,),
        out_shape=jax.ShapeDtypeStruct((nm, 1, 4 * D), jnp.float32),
        in_specs=[pl.BlockSpec((PB, 2 * D), lambda k: (k, 0)),
                  pl.BlockSpec((1, D), lambda k: (0, 0)),
                  pl.BlockSpec((1, D), lambda k: (0, 0))],
        out_specs=pl.BlockSpec((1, 1, 4 * D), lambda k: (k, 0, 0)),
    )(packed, gamma, beta)
    return out.reshape(nm, 4 * D)


def _mat_spec(kk):
    return pl.BlockSpec((RP, 2 * D), lambda i, kk=kk: (kk * NBLK + i, 0))


def _full_spec(shape):
    nd = len(shape)
    return pl.BlockSpec(shape, lambda i: (0,) * nd)


def _loss_a(packed_a, ssa, rads_a):
    return pl.pallas_call(
        _loss_a_kernel,
        grid=(NBLK,),
        out_shape=jax.ShapeDtypeStruct((1, 1), jnp.float32),
        in_specs=[_mat_spec(k) for k in range(NA)] + [
            _full_spec(ssa.shape),
            pl.BlockSpec((NA, RP, 2), lambda i: (0, i, 0)),
        ],
        out_specs=_full_spec((1, 1)),
    )(packed_a, packed_a, packed_a, packed_a, packed_a, ssa, rads_a)


def _loss_b(packed_b, ssb, rads_b, ri3p, ri4p, rel_embed, rad16):
    return pl.pallas_call(
        _loss_b_kernel,
        grid=(NBLK,),
        out_shape=jax.ShapeDtypeStruct((1, 1), jnp.float32),
        in_specs=[_mat_spec(k) for k in range(NB_)] + [
            _full_spec(ssb.shape),
            pl.BlockSpec((NB_, RP, 2), lambda i: (0, i, 0)),
            pl.BlockSpec((RP, 2), lambda i: (i, 0)),
            pl.BlockSpec((RP, 2), lambda i: (i, 0)),
            _full_spec(rel_embed.shape),
            _full_spec(rad16.shape),
        ],
        out_specs=_full_spec((1, 1)),
    )(packed_b, packed_b, packed_b, packed_b, ssb, rads_b,
      ri3p, ri4p, rel_embed, rad16)


def _tc_loss(packed_a, packed_b, rads_a, rads_b, ri3p, ri4p,
             rel_embed, rad16, gamma, beta):
    ssa = _stats(packed_a, NA, gamma, beta)
    la = _loss_a(packed_a, ssa, rads_a)
    ssb = _stats(packed_b, NB_, gamma, beta)
    lb = _loss_b(packed_b, ssb, rads_b, ri3p, ri4p, rel_embed, rad16)
    return la + lb


def kernel(nf1, nf2, nf3, nf4, go_embed, go_rad, rel_embed, bn_gamma, bn_beta):
    nf1 = nf1.astype(jnp.int32)
    nf2 = nf2.astype(jnp.int32)
    nf3 = nf3.astype(jnp.int32)
    nf4 = nf4.astype(jnp.int32)
    nf1t, nf2t, nf3t, nf4t = nf1.T, nf2.T, nf3.T, nf4.T
    go_rad_flat = go_rad.reshape(-1)
    packed_a, rad_a = _sc_gather(go_embed, go_rad_flat,
                                 nf1t, nf2t, nf3t, nf4t, kset=(0, 1, 2, 3, 4))
    packed_b, rad_b = _sc_gather(go_embed, go_rad_flat,
                                 nf1t, nf2t, nf3t, nf4t, kset=(5, 6, 7, 8))
    loss = _tc_loss(packed_a, packed_b,
                    _rad_transpose(rad_a, NA), _rad_transpose(rad_b, NB_),
                    nf3t[0].reshape(2, PB).T, nf4t[1].reshape(2, PB).T,
                    rel_embed, go_rad[:16],
                    bn_gamma.reshape(1, D), bn_beta.reshape(1, D))
    return loss[0, 0]


# speedup vs baseline: 1.0974x; 1.0196x over previous
import functools

import jax
import jax.numpy as jnp
from jax import lax
from jax.experimental import pallas as pl
from jax.experimental.pallas import tpu as pltpu
from jax.experimental.pallas import tpu_sc as plsc

D = 64
B = 16384
NMAT = 9
NIDX = NMAT * B
NW = 32
CHUNK = 128
CPW = NIDX // (NW * CHUNK)
MARGIN = 0.1
EPS = 1e-5


PBC = B // 2


def _sc_gather(go_embed, go_rad_flat, nf1t, nf2t, nf3t, nf4t, kset):
    nm = len(kset)
    src_of = {0: (2, 0), 1: (2, 1), 2: (3, 0), 3: (3, 1), 4: (3, 2),
              5: (4, 1), 6: (4, 2), 7: (5, 0), 8: (5, 2)}

    @functools.partial(
        pl.kernel,
        out_type=[
            jax.ShapeDtypeStruct((nm * PBC, 2 * D), jnp.float32),
            jax.ShapeDtypeStruct((nm * B,), jnp.float32),
        ],
        mesh=plsc.VectorSubcoreMesh(core_axis_name="c", subcore_axis_name="s"),
        compiler_params=pltpu.CompilerParams(use_tc_tiling_on_sc=False),
        scratch_types=[
            pltpu.VMEM((CHUNK,), jnp.int32),
            pltpu.VMEM((CHUNK,), jnp.int32),
            pltpu.VMEM((CHUNK, D), jnp.float32),
            pltpu.VMEM((CHUNK, D), jnp.float32),
            pltpu.VMEM((CHUNK,), jnp.float32),
            pltpu.VMEM((CHUNK,), jnp.float32),
            pltpu.SemaphoreType.DMA,
            pltpu.SemaphoreType.DMA,
            pltpu.SemaphoreType.DMA,
            pltpu.SemaphoreType.DMA,
        ],
    )
    def gather_kernel(emb_hbm, rad_hbm, n1_hbm, n2_hbm, n3_hbm, n4_hbm,
                      rows_out, rad_out,
                      idx0, idx1, rows0, rows1, radv0, radv1,
                      sg0, sg1, sr0, sr1):
        wid = lax.axis_index("s") * 2 + lax.axis_index("c")
        refs = (emb_hbm, rad_hbm, n1_hbm, n2_hbm, n3_hbm, n4_hbm)
        slots = ((idx0, rows0, radv0, sg0, sr0),
                 (idx1, rows1, radv1, sg1, sr1))
        njobs = 4 * nm

        def job(j):
            m, t = j // 4, j % 4
            h, tt = t // 2, t % 2
            return m, h, wid * 2 + tt

        def issue(j, slot):
            idxb, rowsb, radb, sg, sr = slot
            m, h, c = job(j)
            pos, col = src_of[kset[m]]
            pltpu.sync_copy(
                refs[pos].at[col, pl.ds(h * PBC + c * CHUNK, CHUNK)], idxb)
            cp_r = pltpu.async_copy(emb_hbm.at[idxb], rowsb, sg)
            cp_d = pltpu.async_copy(rad_hbm.at[idxb], radb, sr)
            return cp_r, cp_d

        def drain(j, slot, cps):
            idxb, rowsb, radb, sg, sr = slot
            cps[0].wait()
            cps[1].wait()
            m, h, c = job(j)
            pltpu.sync_copy(
                rowsb,
                rows_out.at[pl.ds(m * PBC + c * CHUNK, CHUNK),
                            pl.ds(h * D, D)])
            pltpu.sync_copy(
                radb,
                rad_out.at[pl.ds(m * B + h * PBC + c * CHUNK, CHUNK)])

        cps = issue(0, slots[0])
        for j in range(njobs):
            nxt = issue(j + 1, slots[(j + 1) % 2]) if j + 1 < njobs else None
            drain(j, slots[j % 2], cps)
            cps = nxt

    return gather_kernel(go_embed, go_rad_flat, nf1t, nf2t, nf3t, nf4t)


def _radt_kernel(in_ref, out_ref):
    out_ref[0, :, :] = jnp.transpose(in_ref[0, :, :], (1, 0))


def _rad_transpose(rad_flat, nm):
    return pl.pallas_call(
        _radt_kernel,
        grid=(nm,),
        out_shape=jax.ShapeDtypeStruct((nm, PBC, 2), jnp.float32),
        in_specs=[pl.BlockSpec((1, 2, PBC), lambda k: (k, 0, 0))],
        out_specs=pl.BlockSpec((1, PBC, 2), lambda k: (k, 0, 0)),
    )(rad_flat.reshape(nm, 2, PBC))


PB = B // 2
NBLK = 4
RP = PB // NBLK


def _stats_kernel(p_ref, g_ref, b_ref, out_ref):
    x = p_ref[:, :]
    s = jnp.sum(x, axis=0, keepdims=True)
    q = jnp.sum(x * x, axis=0, keepdims=True)
    sd = s[:, 0:D] + s[:, D:2 * D]
    qd = q[:, 0:D] + q[:, D:2 * D]
    m = sd * (1.0 / B)
    v = qd * (1.0 / B) - m * m
    sc = lax.rsqrt(v + EPS) * g_ref[:, :]
    sh = b_ref[:, :] - m * sc
    sc128 = jnp.concatenate([sc, sc], axis=1)
    sh128 = jnp.concatenate([sh, sh], axis=1)
    out_ref[0, :, :] = jnp.concatenate([sc128, sh128], axis=1)


def _relu(x):
    return jnp.maximum(x, 0.0)


NA = 5
NB_ = NMAT - NA


def _half_w():
    ii = lax.broadcasted_iota(jnp.int32, (128, 2), 0)
    jj = lax.broadcasted_iota(jnp.int32, (128, 2), 1)
    return jnp.where((ii < D) == (jj == 0), 1.0, 0.0).astype(jnp.float32)


def _bnx(mat, ss_ref, kk):
    scsh = ss_ref[kk:kk + 1, :]
    return mat[:, :] * scsh[:, 0:128] + scsh[:, 128:256]


def _accum(out_ref, i, part):
    @pl.when(i == 0)
    def _():
        out_ref[:, :] = jnp.zeros((1, 1), jnp.float32)

    out_ref[:, :] += jnp.reshape(part * (1.0 / B), (1, 1))


def _loss_a_kernel(m0, m1, m2, m3, m4, ss_ref, rads_ref, out_ref):
    i = pl.program_id(0)
    half_w = _half_w()

    def rnorm(dd):
        return jnp.sqrt(jnp.dot(dd * dd, half_w,
                                preferred_element_type=jnp.float32))

    def radp(k):
        return jnp.abs(rads_ref[k, :, :])

    part = jnp.sum(_relu(rnorm(_bnx(m0, ss_ref, 0) - _bnx(m1, ss_ref, 1))
                         + radp(0) - radp(1) - MARGIN))
    c = _bnx(m2, ss_ref, 2)
    d = _bnx(m3, ss_ref, 3)
    e = _bnx(m4, ss_ref, 4)
    rc = radp(2)
    rd = radp(3)
    part += jnp.sum(_relu(rnorm(c - d) - (rc + rd) - MARGIN)
                    + _relu(rnorm(e - c) - rc - MARGIN)
                    + _relu(rnorm(e - d) - rd - MARGIN))
    _accum(out_ref, i, part)


def _loss_b_kernel(m5, m6, m7, m8, ss_ref, rads_ref, ri3_ref, ri4_ref,
                   rel_ref, rad16_ref, out_ref):
    i = pl.program_id(0)
    half_w = _half_w()

    def rnorm(dd):
        return jnp.sqrt(jnp.dot(dd * dd, half_w,
                                preferred_element_type=jnp.float32))

    def radp(k):
        return jnp.abs(rads_ref[k, :, :])

    i16 = lax.broadcasted_iota(jnp.int32, (RP, 16), 1)
    f32 = jnp.float32
    oh3 = jnp.concatenate([(i16 == ri3_ref[:, 0:1]).astype(f32),
                           (i16 == ri3_ref[:, 1:2]).astype(f32)], axis=1)
    oh4 = jnp.concatenate([(i16 == ri4_ref[:, 0:1]).astype(f32),
                           (i16 == ri4_ref[:, 1:2]).astype(f32)], axis=1)
    z = jnp.zeros((16, D), f32)
    relblk = jnp.concatenate(
        [jnp.concatenate([rel_ref[:, :], z], axis=1),
         jnp.concatenate([z, rel_ref[:, :]], axis=1)], axis=0)
    rE3 = jnp.dot(oh3, relblk, preferred_element_type=f32)
    rE4 = jnp.dot(oh4, relblk, preferred_element_type=f32)
    a16 = jnp.abs(rad16_ref[:, :])
    z16 = jnp.zeros((16, 1), f32)
    w4 = jnp.concatenate([jnp.concatenate([a16, z16], axis=0),
                          jnp.concatenate([z16, a16], axis=0)], axis=1)
    rc4 = jnp.dot(oh4, w4, preferred_element_type=f32)

    part = jnp.sum(_relu(rnorm(_bnx(m5, ss_ref, 0) + rE3
                               - _bnx(m6, ss_ref, 1))
                         + radp(0) - radp(1) - MARGIN))
    part += jnp.sum(_relu(rnorm(_bnx(m7, ss_ref, 2)
                                - (_bnx(m8, ss_ref, 3) + rE4))
                          - (rc4 + radp(3)) - MARGIN))
    _accum(out_ref, i, part)


def _stats(packed, nm, gamma, beta):
    out = pl.pallas_call(
        _stats_kernel,
        grid=(nm,),
        out_shape=jax.ShapeDtypeStruct((nm, 1, 4 * D), jnp.float32),
        in_specs=[pl.BlockSpec((PB, 2 * D), lambda k: (k, 0)),
                  pl.BlockSpec((1, D), lambda k: (0, 0)),
                  pl.BlockSpec((1, D), lambda k: (0, 0))],
        out_specs=pl.BlockSpec((1, 1, 4 * D), lambda k: (k, 0, 0)),
    )(packed, gamma, beta)
    return out.reshape(nm, 4 * D)


def _mat_spec(kk):
    return pl.BlockSpec((RP, 2 * D), lambda i, kk=kk: (kk * NBLK + i, 0))


def _full_spec(shape):
    nd = len(shape)
    return pl.BlockSpec(shape, lambda i: (0,) * nd)


def _loss_a(packed_a, ssa, rads_a):
    return pl.pallas_call(
        _loss_a_kernel,
        grid=(NBLK,),
        out_shape=jax.ShapeDtypeStruct((1, 1), jnp.float32),
        in_specs=[_mat_spec(k) for k in range(NA)] + [
            _full_spec(ssa.shape),
            pl.BlockSpec((NA, RP, 2), lambda i: (0, i, 0)),
        ],
        out_specs=_full_spec((1, 1)),
    )(packed_a, packed_a, packed_a, packed_a, packed_a, ssa, rads_a)


def _loss_b(packed_b, ssb, rads_b, ri3p, ri4p, rel_embed, rad16):
    return pl.pallas_call(
        _loss_b_kernel,
        grid=(NBLK,),
        out_shape=jax.ShapeDtypeStruct((1, 1), jnp.float32),
        in_specs=[_mat_spec(k) for k in range(NB_)] + [
            _full_spec(ssb.shape),
            pl.BlockSpec((NB_, RP, 2), lambda i: (0, i, 0)),
            pl.BlockSpec((RP, 2), lambda i: (i, 0)),
            pl.BlockSpec((RP, 2), lambda i: (i, 0)),
            _full_spec(rel_embed.shape),
            _full_spec(rad16.shape),
        ],
        out_specs=_full_spec((1, 1)),
    )(packed_b, packed_b, packed_b, packed_b, ssb, rads_b,
      ri3p, ri4p, rel_embed, rad16)


def _tc_loss(packed_a, packed_b, rads_a, rads_b, ri3p, ri4p,
             rel_embed, rad16, gamma, beta):
    ssa = _stats(packed_a, NA, gamma, beta)
    la = _loss_a(packed_a, ssa, rads_a)
    ssb = _stats(packed_b, NB_, gamma, beta)
    lb = _loss_b(packed_b, ssb, rads_b, ri3p, ri4p, rel_embed, rad16)
    return la + lb


def kernel(nf1, nf2, nf3, nf4, go_embed, go_rad, rel_embed, bn_gamma, bn_beta):
    nf1 = nf1.astype(jnp.int32)
    nf2 = nf2.astype(jnp.int32)
    nf3 = nf3.astype(jnp.int32)
    nf4 = nf4.astype(jnp.int32)
    nf1t, nf2t, nf3t, nf4t = nf1.T, nf2.T, nf3.T, nf4.T
    go_rad_flat = go_rad.reshape(-1)
    packed_a, rad_a = _sc_gather(go_embed, go_rad_flat,
                                 nf1t, nf2t, nf3t, nf4t, kset=(0, 1, 2, 3, 4))
    packed_b, rad_b = _sc_gather(go_embed, go_rad_flat,
                                 nf1t, nf2t, nf3t, nf4t, kset=(5, 6, 7, 8))
    loss = _tc_loss(packed_a, packed_b,
                    _rad_transpose(rad_a, NA), _rad_transpose(rad_b, NB_),
                    nf3t[0].reshape(2, PB).T, nf4t[1].reshape(2, PB).T,
                    rel_embed, go_rad[:16],
                    bn_gamma.reshape(1, D), bn_beta.reshape(1, D))
    return loss[0, 0]
